# baseline (speedup 1.0000x reference)
import dataclasses
import functools

import jax
import jax.numpy as jnp
from jax import lax
from jax.experimental import pallas as pl
from jax.experimental.pallas import tpu as pltpu
from jax.experimental.pallas import tpu_sc as plsc

N = 10000
D = 128
E = 320000
ALPHA = 0.1
BETA1 = 0.6931471805599453
BETA2 = 0.4054651081081644

NC, NS = 2, 16
K = 128
CH = 160
EPW = CH * K
EPAD = NS * EPW
NPAD = 10240
RPS = NPAD // NS
ZR = RPS // 2
NBUF = 2
T = 40
NT = CH // T


@functools.cache
def _mesh():
    return plsc.VectorSubcoreMesh(core_axis_name="c", subcore_axis_name="s",
                                  num_cores=NC, num_subcores=NS)


def _sc_hist(idx2, zn):
    @functools.partial(
        pl.kernel,
        out_type=jax.ShapeDtypeStruct((2, NC, NS, NPAD), jnp.float32),
        mesh=_mesh(),
        scratch_types=[
            pltpu.VMEM((CH, K), jnp.int32),
            pltpu.VMEM((NPAD,), jnp.float32),
        ],
        compiler_params=dataclasses.replace(
            pltpu.CompilerParams(), needs_layout_passes=False),
    )
    def hist_k(idx_hbm, zn_hbm, out_hbm, idx_v, cnt_v):
        c = lax.axis_index("c")
        s = lax.axis_index("s")
        ones16 = jnp.ones((16,), jnp.float32)

        for p in range(2):
            pltpu.sync_copy(idx_hbm.at[p].at[c].at[s], idx_v)
            pltpu.sync_copy(zn_hbm, cnt_v)

            @pl.loop(0, CH)
            def _row(j):
                for l in range(K // 16):
                    idx16 = idx_v[j, pl.ds(l * 16, 16)]
                    plsc.addupdate_scatter(cnt_v, [idx16], ones16)

            pltpu.sync_copy(cnt_v, out_hbm.at[p].at[c].at[s])

    return hist_k(idx2, zn)


def _sc_agg(xs, srcc, dstc, zrows):
    @functools.partial(
        pl.kernel,
        out_type=jax.ShapeDtypeStruct((NC, NPAD, D), jnp.float32),
        mesh=_mesh(),
        scratch_types=[
            pltpu.VMEM((T, K), jnp.int32),
            pltpu.VMEM((T, K), jnp.int32),
            pltpu.VMEM((K, D), jnp.float32),
            pltpu.VMEM((K, D), jnp.float32),
            pltpu.VMEM_SHARED((NPAD, D), jnp.float32),
            pltpu.SemaphoreType.DMA,
            pltpu.SemaphoreType.DMA,
            pltpu.SemaphoreType.DMA,
            pltpu.SemaphoreType.DMA,
        ],
    )
    def agg_k(xs_hbm, src_hbm, dst_hbm, z_hbm, out_hbm,
              src_v, dst_v, b0, b1, acc_sh, g0, g1, c0, c1):
        bufs = (b0, b1)
        gsems = (g0, g1)
        csems = (c0, c1)
        c = lax.axis_index("c")
        s = lax.axis_index("s")
        table = xs_hbm.at[c]
        pltpu.sync_copy(z_hbm, acc_sh.at[pl.ds(s * RPS, ZR)])
        pltpu.sync_copy(z_hbm, acc_sh.at[pl.ds(s * RPS + ZR, ZR)])
        plsc.subcore_barrier()

        @pl.loop(0, NT)
        def _tile(nt):
            pltpu.sync_copy(src_hbm.at[c].at[s].at[pl.ds(nt * T, T)], src_v)
            pltpu.sync_copy(dst_hbm.at[c].at[s].at[pl.ds(nt * T, T)], dst_v)
            for b in range(NBUF):
                pltpu.async_copy(table.at[src_v.at[b]], bufs[b], gsems[b])

            @pl.loop(0, T // NBUF)
            def _chunks(t):
                j0 = NBUF * t
                for b in range(NBUF):
                    j = j0 + b
                    pltpu.make_async_copy(
                        table.at[src_v.at[j]], bufs[b], gsems[b]).wait()
                    pltpu.async_copy(bufs[b], acc_sh.at[dst_v.at[j]],
                                     csems[b], add=True)
                for b in range(NBUF):
                    j = j0 + b

                    def _prefetch(b=b, j=j):
                        pltpu.make_async_copy(
                            bufs[b], acc_sh.at[dst_v.at[j]], csems[b]).wait()
                        pltpu.async_copy(
                            table.at[src_v.at[j + NBUF]], bufs[b], gsems[b])

                    pl.when(j + NBUF < T)(_prefetch)

            for b in range(NBUF):
                pltpu.make_async_copy(
                    bufs[b], acc_sh.at[dst_v.at[T - NBUF + b]],
                    csems[b]).wait()

        plsc.subcore_barrier()
        pltpu.sync_copy(acc_sh.at[pl.ds(s * RPS, RPS)],
                        out_hbm.at[c].at[pl.ds(s * RPS, RPS)])

    return agg_k(xs, srcc, dstc, zrows)


BN = 1024
GRID = NPAD // BN


def _tc_prep(hist, f0):
    def body(h_ref, f0_ref, ns_ref, nd_ref, xs_ref):
        ones_col = jnp.ones((NS, 1), jnp.float32)
        for g in range(NC):
            degs = lax.dot_general(
                h_ref[0, g], ones_col, (((0,), (0,)), ((), ())),
                preferred_element_type=jnp.float32,
                precision=lax.Precision.HIGHEST)
            degd = lax.dot_general(
                h_ref[1, g], ones_col, (((0,), (0,)), ((), ())),
                preferred_element_type=jnp.float32,
                precision=lax.Precision.HIGHEST)
            ns = jnp.where(degs > 0.0, lax.rsqrt(degs), 0.0)
            nd = jnp.where(degd > 0.0, lax.rsqrt(degd), 0.0)
            ns_ref[g] = ns
            nd_ref[g] = nd
            xs_ref[g] = f0_ref[g] * ns

    return pl.pallas_call(
        body,
        grid=(GRID,),
        in_specs=[
            pl.BlockSpec((2, NC, NS, BN), lambda i: (0, 0, 0, i)),
            pl.BlockSpec((NC, BN, D), lambda i: (0, i, 0)),
        ],
        out_specs=[
            pl.BlockSpec((NC, BN, 1), lambda i: (0, i, 0)),
            pl.BlockSpec((NC, BN, 1), lambda i: (0, i, 0)),
            pl.BlockSpec((NC, BN, D), lambda i: (0, i, 0)),
        ],
        out_shape=[
            jax.ShapeDtypeStruct((NC, NPAD, 1), jnp.float32),
            jax.ShapeDtypeStruct((NC, NPAD, 1), jnp.float32),
            jax.ShapeDtypeStruct((NC, NPAD, D), jnp.float32),
        ],
    )(hist, f0)


def _tc_layer1(p, nd, ns, f0, W):
    def body(p_ref, nd_ref, ns_ref, f0_ref, w_ref, x_ref, xs_ref):
        for g in range(NC):
            agg = p_ref[g] * nd_ref[g]
            feat = (1.0 - ALPHA) * agg + ALPHA * f0_ref[g]
            rst = (1.0 - BETA1) * feat + BETA1 * jnp.dot(
                feat, w_ref[...], preferred_element_type=jnp.float32,
                precision=lax.Precision.HIGHEST)
            x = jnp.maximum(rst, 0.0)
            x_ref[g] = x
            xs_ref[g] = x * ns_ref[g]

    return pl.pallas_call(
        body,
        grid=(GRID,),
        in_specs=[
            pl.BlockSpec((NC, BN, D), lambda i: (0, i, 0)),
            pl.BlockSpec((NC, BN, 1), lambda i: (0, i, 0)),
            pl.BlockSpec((NC, BN, 1), lambda i: (0, i, 0)),
            pl.BlockSpec((NC, BN, D), lambda i: (0, i, 0)),
            pl.BlockSpec((D, D), lambda i: (0, 0)),
        ],
        out_specs=[pl.BlockSpec((NC, BN, D), lambda i: (0, i, 0))] * 2,
        out_shape=[jax.ShapeDtypeStruct((NC, NPAD, D), jnp.float32)] * 2,
    )(p, nd, ns, f0, W)


def _tc_layer2(p, nd, f0, W):
    def body(p_ref, nd_ref, f0_ref, w_ref, h_ref, st_ref):
        @pl.when(pl.program_id(0) == 0)
        def _():
            st_ref[...] = jnp.zeros((NC, 8, D), jnp.float32)

        rid = lax.broadcasted_iota(jnp.int32, (8, D), 0)
        for g in range(NC):
            agg = p_ref[g] * nd_ref[g]
            feat = (1.0 - ALPHA) * agg + ALPHA * f0_ref[g]
            rst = (1.0 - BETA2) * feat + BETA2 * jnp.dot(
                feat, w_ref[...], preferred_element_type=jnp.float32,
                precision=lax.Precision.HIGHEST)
            x = jnp.maximum(rst, 0.0)
            h_ref[g] = x
            s1 = jnp.sum(x, axis=0, keepdims=True)
            s2 = jnp.sum(x * x, axis=0, keepdims=True)
            st_ref[g] += jnp.where(rid == 0, s1, 0.0) + jnp.where(rid == 1, s2, 0.0)

    return pl.pallas_call(
        body,
        grid=(GRID,),
        in_specs=[
            pl.BlockSpec((NC, BN, D), lambda i: (0, i, 0)),
            pl.BlockSpec((NC, BN, 1), lambda i: (0, i, 0)),
            pl.BlockSpec((NC, BN, D), lambda i: (0, i, 0)),
            pl.BlockSpec((D, D), lambda i: (0, 0)),
        ],
        out_specs=[
            pl.BlockSpec((NC, BN, D), lambda i: (0, i, 0)),
            pl.BlockSpec((NC, 8, D), lambda i: (0, 0, 0)),
        ],
        out_shape=[
            jax.ShapeDtypeStruct((NC, NPAD, D), jnp.float32),
            jax.ShapeDtypeStruct((NC, 8, D), jnp.float32),
        ],
    )(p, nd, f0, W)


def _tc_std(h, st):
    def body(h_ref, st_ref, z_ref):
        for g in range(NC):
            s1 = st_ref[g, 0:1, :]
            s2 = st_ref[g, 1:2, :]
            mean = s1 * (1.0 / N)
            var = (s2 - (mean * mean) * N) * (1.0 / (N - 1))
            sd = jnp.sqrt(jnp.maximum(var, 0.0))
            inv = 1.0 / jnp.maximum(sd, 1e-12)
            z_ref[g] = (h_ref[g] - mean) * inv

    return pl.pallas_call(
        body,
        grid=(GRID,),
        in_specs=[
            pl.BlockSpec((NC, BN, D), lambda i: (0, i, 0)),
            pl.BlockSpec((NC, 8, D), lambda i: (0, 0, 0)),
        ],
        out_specs=pl.BlockSpec((NC, BN, D), lambda i: (0, i, 0)),
        out_shape=jax.ShapeDtypeStruct((NC, NPAD, D), jnp.float32),
    )(h, st)


def kernel(feat1, edge_index1, feat2, edge_index2, W1, W2):
    f0 = jnp.stack([
        jnp.pad(feat1, ((0, NPAD - N), (0, 0))),
        jnp.pad(feat2, ((0, NPAD - N), (0, 0))),
    ])

    def chunk(idx):
        pad = jnp.full((EPAD - E,), N, jnp.int32)
        return jnp.concatenate([idx.astype(jnp.int32), pad]).reshape(NS, CH, K)

    srcc = jnp.stack([chunk(edge_index1[0]), chunk(edge_index2[0])])
    dstc = jnp.stack([chunk(edge_index1[1]), chunk(edge_index2[1])])
    idx2 = jnp.stack([srcc, dstc])
    zn = jnp.zeros((NPAD,), jnp.float32)
    zrows = jnp.zeros((ZR, D), jnp.float32)

    hist = _sc_hist(idx2, zn)
    ns, nd, xs = _tc_prep(hist, f0)

    p = _sc_agg(xs, srcc, dstc, zrows)
    x, xsb = _tc_layer1(p, nd, ns, f0, W1)
    q = _sc_agg(xsb, srcc, dstc, zrows)
    h, st = _tc_layer2(q, nd, f0, W2)
    z = _tc_std(h, st)
    return z[0, :N], z[1, :N]


# speedup vs baseline: 1.0760x; 1.0760x over previous
import dataclasses
import functools

import jax
import jax.numpy as jnp
from jax import lax
from jax.experimental import pallas as pl
from jax.experimental.pallas import tpu as pltpu
from jax.experimental.pallas import tpu_sc as plsc

N = 10000
D = 128
E = 320000
ALPHA = 0.1
BETA1 = 0.6931471805599453
BETA2 = 0.4054651081081644

NC, NS = 2, 16
K = 128
CH = 160
EPW = CH * K
EPAD = NS * EPW
NPAD = 10240
RPS = NPAD // NS
ZR = RPS // 2
NBUF = 2
T = 40
NT = CH // T


@functools.cache
def _mesh():
    return plsc.VectorSubcoreMesh(core_axis_name="c", subcore_axis_name="s",
                                  num_cores=NC, num_subcores=NS)


def _sc_hist(idx2, zn):
    @functools.partial(
        pl.kernel,
        out_type=jax.ShapeDtypeStruct((2, NC, NS, NPAD), jnp.float32),
        mesh=_mesh(),
        scratch_types=[
            pltpu.VMEM((CH, K), jnp.int32),
            pltpu.VMEM((NPAD,), jnp.float32),
        ],
        compiler_params=dataclasses.replace(
            pltpu.CompilerParams(), needs_layout_passes=False),
    )
    def hist_k(idx_hbm, zn_hbm, out_hbm, idx_v, cnt_v):
        c = lax.axis_index("c")
        s = lax.axis_index("s")
        ones16 = jnp.ones((16,), jnp.float32)

        for p in range(2):
            pltpu.sync_copy(idx_hbm.at[p].at[c].at[s], idx_v)
            pltpu.sync_copy(zn_hbm, cnt_v)

            @pl.loop(0, CH)
            def _row(j):
                for l in range(K // 16):
                    idx16 = idx_v[j, pl.ds(l * 16, 16)]
                    plsc.addupdate_scatter(cnt_v, [idx16], ones16)

            pltpu.sync_copy(cnt_v, out_hbm.at[p].at[c].at[s])

    return hist_k(idx2, zn)


def _sc_agg(xs, srcc, dstc, zrows):
    @functools.partial(
        pl.kernel,
        out_type=jax.ShapeDtypeStruct((NC, NPAD, D), jnp.float32),
        mesh=_mesh(),
        scratch_types=[
            pltpu.VMEM((T, K), jnp.int32),
            pltpu.VMEM((T, K), jnp.int32),
            pltpu.VMEM((K, D), jnp.float32),
            pltpu.VMEM((K, D), jnp.float32),
            pltpu.VMEM_SHARED((NPAD, D), jnp.float32),
            pltpu.SemaphoreType.DMA,
            pltpu.SemaphoreType.DMA,
        ],
    )
    def agg_k(xs_hbm, src_hbm, dst_hbm, z_hbm, out_hbm,
              src_v, dst_v, b0, b1, acc_sh, g0, g1):
        bufs = (b0, b1)
        gsems = (g0, g1)
        c = lax.axis_index("c")
        s = lax.axis_index("s")
        table = xs_hbm.at[c]
        pltpu.sync_copy(z_hbm, acc_sh.at[pl.ds(s * RPS, ZR)])
        pltpu.sync_copy(z_hbm, acc_sh.at[pl.ds(s * RPS + ZR, ZR)])
        plsc.subcore_barrier()

        @pl.loop(0, NT)
        def _tile(nt):
            pltpu.sync_copy(src_hbm.at[c].at[s].at[pl.ds(nt * T, T)], src_v)
            pltpu.sync_copy(dst_hbm.at[c].at[s].at[pl.ds(nt * T, T)], dst_v)
            for b in range(NBUF):
                pltpu.async_copy(table.at[src_v.at[b]], bufs[b], gsems[b])

            @pl.loop(0, T // NBUF)
            def _chunks(t):
                j0 = NBUF * t
                for b in range(NBUF):
                    j = j0 + b
                    pltpu.make_async_copy(
                        table.at[src_v.at[j]], bufs[b], gsems[b]).wait()
                    pltpu.sync_copy(bufs[b], acc_sh.at[dst_v.at[j]], add=True)

                    def _prefetch(b=b, j=j):
                        pltpu.async_copy(
                            table.at[src_v.at[j + NBUF]], bufs[b], gsems[b])

                    pl.when(j + NBUF < T)(_prefetch)

        plsc.subcore_barrier()
        pltpu.sync_copy(acc_sh.at[pl.ds(s * RPS, RPS)],
                        out_hbm.at[c].at[pl.ds(s * RPS, RPS)])

    return agg_k(xs, srcc, dstc, zrows)


BN = 1024
GRID = NPAD // BN


def _tc_prep(hist, f0):
    def body(h_ref, f0_ref, ns_ref, nd_ref, xs_ref):
        ones_col = jnp.ones((NS, 1), jnp.float32)
        for g in range(NC):
            degs = lax.dot_general(
                h_ref[0, g], ones_col, (((0,), (0,)), ((), ())),
                preferred_element_type=jnp.float32,
                precision=lax.Precision.HIGHEST)
            degd = lax.dot_general(
                h_ref[1, g], ones_col, (((0,), (0,)), ((), ())),
                preferred_element_type=jnp.float32,
                precision=lax.Precision.HIGHEST)
            ns = jnp.where(degs > 0.0, lax.rsqrt(degs), 0.0)
            nd = jnp.where(degd > 0.0, lax.rsqrt(degd), 0.0)
            ns_ref[g] = ns
            nd_ref[g] = nd
            xs_ref[g] = f0_ref[g] * ns

    return pl.pallas_call(
        body,
        grid=(GRID,),
        in_specs=[
            pl.BlockSpec((2, NC, NS, BN), lambda i: (0, 0, 0, i)),
            pl.BlockSpec((NC, BN, D), lambda i: (0, i, 0)),
        ],
        out_specs=[
            pl.BlockSpec((NC, BN, 1), lambda i: (0, i, 0)),
            pl.BlockSpec((NC, BN, 1), lambda i: (0, i, 0)),
            pl.BlockSpec((NC, BN, D), lambda i: (0, i, 0)),
        ],
        out_shape=[
            jax.ShapeDtypeStruct((NC, NPAD, 1), jnp.float32),
            jax.ShapeDtypeStruct((NC, NPAD, 1), jnp.float32),
            jax.ShapeDtypeStruct((NC, NPAD, D), jnp.float32),
        ],
    )(hist, f0)


def _tc_layer1(p, nd, ns, f0, W):
    def body(p_ref, nd_ref, ns_ref, f0_ref, w_ref, x_ref, xs_ref):
        for g in range(NC):
            agg = p_ref[g] * nd_ref[g]
            feat = (1.0 - ALPHA) * agg + ALPHA * f0_ref[g]
            rst = (1.0 - BETA1) * feat + BETA1 * jnp.dot(
                feat, w_ref[...], preferred_element_type=jnp.float32,
                precision=lax.Precision.HIGHEST)
            x = jnp.maximum(rst, 0.0)
            x_ref[g] = x
            xs_ref[g] = x * ns_ref[g]

    return pl.pallas_call(
        body,
        grid=(GRID,),
        in_specs=[
            pl.BlockSpec((NC, BN, D), lambda i: (0, i, 0)),
            pl.BlockSpec((NC, BN, 1), lambda i: (0, i, 0)),
            pl.BlockSpec((NC, BN, 1), lambda i: (0, i, 0)),
            pl.BlockSpec((NC, BN, D), lambda i: (0, i, 0)),
            pl.BlockSpec((D, D), lambda i: (0, 0)),
        ],
        out_specs=[pl.BlockSpec((NC, BN, D), lambda i: (0, i, 0))] * 2,
        out_shape=[jax.ShapeDtypeStruct((NC, NPAD, D), jnp.float32)] * 2,
    )(p, nd, ns, f0, W)


def _tc_layer2(p, nd, f0, W):
    def body(p_ref, nd_ref, f0_ref, w_ref, h_ref, st_ref):
        @pl.when(pl.program_id(0) == 0)
        def _():
            st_ref[...] = jnp.zeros((NC, 8, D), jnp.float32)

        rid = lax.broadcasted_iota(jnp.int32, (8, D), 0)
        for g in range(NC):
            agg = p_ref[g] * nd_ref[g]
            feat = (1.0 - ALPHA) * agg + ALPHA * f0_ref[g]
            rst = (1.0 - BETA2) * feat + BETA2 * jnp.dot(
                feat, w_ref[...], preferred_element_type=jnp.float32,
                precision=lax.Precision.HIGHEST)
            x = jnp.maximum(rst, 0.0)
            h_ref[g] = x
            s1 = jnp.sum(x, axis=0, keepdims=True)
            s2 = jnp.sum(x * x, axis=0, keepdims=True)
            st_ref[g] += jnp.where(rid == 0, s1, 0.0) + jnp.where(rid == 1, s2, 0.0)

    return pl.pallas_call(
        body,
        grid=(GRID,),
        in_specs=[
            pl.BlockSpec((NC, BN, D), lambda i: (0, i, 0)),
            pl.BlockSpec((NC, BN, 1), lambda i: (0, i, 0)),
            pl.BlockSpec((NC, BN, D), lambda i: (0, i, 0)),
            pl.BlockSpec((D, D), lambda i: (0, 0)),
        ],
        out_specs=[
            pl.BlockSpec((NC, BN, D), lambda i: (0, i, 0)),
            pl.BlockSpec((NC, 8, D), lambda i: (0, 0, 0)),
        ],
        out_shape=[
            jax.ShapeDtypeStruct((NC, NPAD, D), jnp.float32),
            jax.ShapeDtypeStruct((NC, 8, D), jnp.float32),
        ],
    )(p, nd, f0, W)


def _tc_std(h, st):
    def body(h_ref, st_ref, z_ref):
        for g in range(NC):
            s1 = st_ref[g, 0:1, :]
            s2 = st_ref[g, 1:2, :]
            mean = s1 * (1.0 / N)
            var = (s2 - (mean * mean) * N) * (1.0 / (N - 1))
            sd = jnp.sqrt(jnp.maximum(var, 0.0))
            inv = 1.0 / jnp.maximum(sd, 1e-12)
            z_ref[g] = (h_ref[g] - mean) * inv

    return pl.pallas_call(
        body,
        grid=(GRID,),
        in_specs=[
            pl.BlockSpec((NC, BN, D), lambda i: (0, i, 0)),
            pl.BlockSpec((NC, 8, D), lambda i: (0, 0, 0)),
        ],
        out_specs=pl.BlockSpec((NC, BN, D), lambda i: (0, i, 0)),
        out_shape=jax.ShapeDtypeStruct((NC, NPAD, D), jnp.float32),
    )(h, st)


def kernel(feat1, edge_index1, feat2, edge_index2, W1, W2):
    f0 = jnp.stack([
        jnp.pad(feat1, ((0, NPAD - N), (0, 0))),
        jnp.pad(feat2, ((0, NPAD - N), (0, 0))),
    ])

    def chunk(idx):
        pad = jnp.full((EPAD - E,), N, jnp.int32)
        return jnp.concatenate([idx.astype(jnp.int32), pad]).reshape(NS, CH, K)

    srcc = jnp.stack([chunk(edge_index1[0]), chunk(edge_index2[0])])
    dstc = jnp.stack([chunk(edge_index1[1]), chunk(edge_index2[1])])
    idx2 = jnp.stack([srcc, dstc])
    zn = jnp.zeros((NPAD,), jnp.float32)
    zrows = jnp.zeros((ZR, D), jnp.float32)

    hist = _sc_hist(idx2, zn)
    ns, nd, xs = _tc_prep(hist, f0)

    p = _sc_agg(xs, srcc, dstc, zrows)
    x, xsb = _tc_layer1(p, nd, ns, f0, W1)
    q = _sc_agg(xsb, srcc, dstc, zrows)
    h, st = _tc_layer2(q, nd, f0, W2)
    z = _tc_std(h, st)
    return z[0, :N], z[1, :N]
